# baseline (speedup 1.0000x reference)
import functools

import jax
import jax.numpy as jnp
from jax import lax
from jax.experimental import pallas as pl
from jax.experimental.pallas import tpu as pltpu
from jax.experimental.pallas import tpu_sc as plsc

DIM = 64
CAT = 32
NC, NS = 2, 16
NW = NC * NS
CHUNK = 512
SUB = 128


def _sc_gather(node_emb, ctx_emb, nidx, cidx):
    dim = node_emb.shape[1]
    NB = nidx.shape[0]
    CB = cidx.shape[0]
    nb_per = NB // NW
    cb_per = CB // NW
    assert NB % NW == 0 and CB % NW == 0
    assert nb_per % CHUNK == 0 and cb_per % CHUNK == 0

    mesh = plsc.VectorSubcoreMesh(
        core_axis_name="c", subcore_axis_name="s",
        num_cores=NC, num_subcores=NS)

    @functools.partial(
        pl.kernel,
        mesh=mesh,
        out_type=(
            jax.ShapeDtypeStruct((NB, dim), jnp.float32),
            jax.ShapeDtypeStruct((CB, dim), jnp.float32),
        ),
        scratch_types=[
            pltpu.VMEM((CHUNK,), jnp.int32),
            pltpu.VMEM((CHUNK, dim), jnp.float32),
            pltpu.SemaphoreType.DMA,
        ],
        compiler_params=pltpu.CompilerParams(use_tc_tiling_on_sc=False),
    )
    def gather_kernel(node_hbm, ctx_hbm, nidx_hbm, cidx_hbm,
                      nout_hbm, cout_hbm, idx_v, rows_v, sem):
        wid = lax.axis_index("s") * NC + lax.axis_index("c")

        def run(table_hbm, idx_hbm, out_hbm, base, nchunks):
            def body(i, carry):
                off = base + i * CHUNK
                pltpu.sync_copy(idx_hbm.at[pl.ds(off, CHUNK)], idx_v)
                descs = [
                    pltpu.async_copy(
                        table_hbm.at[idx_v.at[pl.ds(k * SUB, SUB)]],
                        rows_v.at[pl.ds(k * SUB, SUB)], sem)
                    for k in range(CHUNK // SUB)
                ]
                for d in descs:
                    d.wait()
                pltpu.sync_copy(rows_v, out_hbm.at[pl.ds(off, CHUNK)])
                return carry
            lax.fori_loop(0, nchunks, body, 0)

        run(node_hbm, nidx_hbm, nout_hbm, wid * nb_per, nb_per // CHUNK)
        run(ctx_hbm, cidx_hbm, cout_hbm, wid * cb_per, cb_per // CHUNK)

    return gather_kernel(node_emb, ctx_emb, nidx, cidx)


def _softmax(x):
    m = jnp.max(x, axis=1, keepdims=True)
    e = jnp.exp(x - m)
    return e / jnp.sum(e, axis=1, keepdims=True)


def _logsig(x):
    return jnp.minimum(x, 0.0) - jnp.log(1.0 + jnp.exp(-jnp.abs(x)))


def _dense_tc(w_e, c_e, c_ctx, negT, g, tau, W_comm):
    B = w_e.shape[0]
    nneg = negT.shape[0]
    R = 1024
    grid = B // R
    dn = (((1,), (1,)), ((), ()))

    def body(w_ref, c_ref, cc_ref, neg_ref, g_ref, tau_ref, W_ref,
             S_ref, P_ref, L_ref):
        W = W_ref[...]
        w_e = w_ref[...]
        pw = w_e * c_ref[...]
        q = lax.dot_general(pw, W, dn, preferred_element_type=jnp.float32)
        S_ref[...] = _softmax(q)
        p = lax.dot_general(w_e, W, dn, preferred_element_type=jnp.float32)
        P_ref[...] = _softmax(p)

        a = (q + g_ref[...]) / tau_ref[0, 0]
        m = jnp.max(a, axis=1, keepdims=True)
        iota = lax.broadcasted_iota(jnp.int32, a.shape, 1)
        ksel = jnp.min(jnp.where(a >= m, iota, CAT), axis=1, keepdims=True)
        H = (iota == ksel).astype(jnp.float32)

        Cc = lax.dot_general(cc_ref[...], W, dn,
                             preferred_element_type=jnp.float32)
        pos = _logsig(jnp.sum(H * Cc, axis=1, keepdims=True))
        V = lax.dot_general(H, W, (((1,), (0,)), ((), ())),
                            preferred_element_type=jnp.float32)
        acc = jnp.zeros((R, 1), jnp.float32)
        for j in range(nneg):
            s = jnp.sum(neg_ref[j] * V, axis=1, keepdims=True)
            acc = acc + _logsig(-s)
        L_ref[0, 0, 0] = jnp.sum(pos + acc * (1.0 / nneg))

    return pl.pallas_call(
        body,
        grid=(grid,),
        in_specs=[
            pl.BlockSpec((R, DIM), lambda i: (i, 0)),
            pl.BlockSpec((R, DIM), lambda i: (i, 0)),
            pl.BlockSpec((R, DIM), lambda i: (i, 0)),
            pl.BlockSpec((nneg, R, DIM), lambda i: (0, i, 0)),
            pl.BlockSpec((R, CAT), lambda i: (i, 0)),
            pl.BlockSpec(memory_space=pltpu.SMEM),
            pl.BlockSpec((CAT, DIM), lambda i: (0, 0)),
        ],
        out_specs=[
            pl.BlockSpec((R, CAT), lambda i: (i, 0)),
            pl.BlockSpec((R, CAT), lambda i: (i, 0)),
            pl.BlockSpec((1, 1, 1), lambda i: (i, 0, 0),
                         memory_space=pltpu.SMEM),
        ],
        out_shape=[
            jax.ShapeDtypeStruct((B, CAT), jnp.float32),
            jax.ShapeDtypeStruct((B, CAT), jnp.float32),
            jax.ShapeDtypeStruct((grid, 1, 1), jnp.float32),
        ],
    )(w_e, c_e, c_ctx, negT, g, tau, W_comm)


def kernel(w, c, neg, temp, node_emb, ctx_emb, W_comm):
    B = w.shape[0]
    nneg = neg.shape[1]
    nidx = jnp.concatenate([w, c])
    cidx = jnp.concatenate([c, neg.T.reshape(-1)])
    node_rows, ctx_rows = _sc_gather(node_emb, ctx_emb, nidx, cidx)
    w_e = node_rows[:B]
    c_e = node_rows[B:]
    c_ctx = ctx_rows[:B]
    negT = ctx_rows[B:].reshape(nneg, B, DIM)

    u = jax.random.uniform(jax.random.key(42), (B, CAT),
                           minval=1e-10, maxval=1.0)
    g = -jnp.log(-jnp.log(u))
    tau = jnp.asarray(temp, jnp.float32).reshape(1, 1)

    S, P, Lp = _dense_tc(w_e, c_e, c_ctx, negT, g, tau, W_comm)
    loss = -jnp.sum(Lp) / B
    return (loss, S, P)


# speedup vs baseline: 1.0143x; 1.0143x over previous
import functools

import jax
import jax.numpy as jnp
from jax import lax
from jax.experimental import pallas as pl
from jax.experimental.pallas import tpu as pltpu
from jax.experimental.pallas import tpu_sc as plsc

DIM = 64
CAT = 32
NC, NS = 2, 16
NW = NC * NS
CHUNK = 512
SUB = 128


def _sc_gather(node_emb, ctx_emb, nidx, cidx):
    dim = node_emb.shape[1]
    NB = nidx.shape[0]
    CB = cidx.shape[0]
    nb_per = NB // NW
    cb_per = CB // NW
    NCHUNK = 512
    CCHUNK = 768
    BUF = max(NCHUNK, CCHUNK)
    assert NB % NW == 0 and CB % NW == 0
    assert nb_per % NCHUNK == 0 and cb_per % CCHUNK == 0

    mesh = plsc.VectorSubcoreMesh(
        core_axis_name="c", subcore_axis_name="s",
        num_cores=NC, num_subcores=NS)

    @functools.partial(
        pl.kernel,
        mesh=mesh,
        out_type=(
            jax.ShapeDtypeStruct((NB, dim), jnp.float32),
            jax.ShapeDtypeStruct((CB, dim), jnp.float32),
        ),
        scratch_types=[
            pltpu.VMEM((nb_per,), jnp.int32),
            pltpu.VMEM((cb_per,), jnp.int32),
            pltpu.VMEM((2, BUF, dim), jnp.float32),
            pltpu.SemaphoreType.DMA((2,)),
            pltpu.SemaphoreType.DMA((2,)),
        ],
        compiler_params=pltpu.CompilerParams(use_tc_tiling_on_sc=False),
    )
    def gather_kernel(node_hbm, ctx_hbm, nidx_hbm, cidx_hbm,
                      nout_hbm, cout_hbm, nidx_v, cidx_v, rows_v,
                      gsem, wsem):
        wid = lax.axis_index("s") * NC + lax.axis_index("c")
        pltpu.sync_copy(nidx_hbm.at[pl.ds(wid * nb_per, nb_per)], nidx_v)
        pltpu.sync_copy(cidx_hbm.at[pl.ds(wid * cb_per, cb_per)], cidx_v)

        def run(table_hbm, idx_ref, out_hbm, base, chunk, nchunks):
            pend_g = [None, None]
            pend_w = [None, None]

            def fire(i):
                p = i % 2
                pend_g[p] = [
                    pltpu.async_copy(
                        table_hbm.at[idx_ref.at[
                            pl.ds(i * chunk + k * SUB, SUB)]],
                        rows_v.at[p].at[pl.ds(k * SUB, SUB)],
                        gsem.at[p])
                    for k in range(chunk // SUB)
                ]

            fire(0)
            for i in range(nchunks):
                if i + 1 < nchunks:
                    p1 = (i + 1) % 2
                    if pend_w[p1] is not None:
                        pend_w[p1].wait()
                        pend_w[p1] = None
                    fire(i + 1)
                p = i % 2
                for d in pend_g[p]:
                    d.wait()
                pend_w[p] = pltpu.async_copy(
                    rows_v.at[p].at[pl.ds(0, chunk)],
                    out_hbm.at[pl.ds(base + i * chunk, chunk)], wsem.at[p])
            for p in range(2):
                if pend_w[p] is not None:
                    pend_w[p].wait()

        run(node_hbm, nidx_v, nout_hbm, wid * nb_per, NCHUNK, nb_per // NCHUNK)
        run(ctx_hbm, cidx_v, cout_hbm, wid * cb_per, CCHUNK, cb_per // CCHUNK)

    return gather_kernel(node_emb, ctx_emb, nidx, cidx)


def _softmax(x):
    m = jnp.max(x, axis=1, keepdims=True)
    e = jnp.exp(x - m)
    return e / jnp.sum(e, axis=1, keepdims=True)


def _logsig(x):
    return jnp.minimum(x, 0.0) - jnp.log(1.0 + jnp.exp(-jnp.abs(x)))


def _dense_tc(w_e, c_e, c_ctx, negT, g, tau, W_comm):
    B = w_e.shape[0]
    nneg = negT.shape[0]
    R = 1024
    grid = B // R
    dn = (((1,), (1,)), ((), ()))

    def body(w_ref, c_ref, cc_ref, neg_ref, g_ref, tau_ref, W_ref,
             S_ref, P_ref, L_ref):
        W = W_ref[...]
        w_e = w_ref[...]
        pw = w_e * c_ref[...]
        q = lax.dot_general(pw, W, dn, preferred_element_type=jnp.float32)
        S_ref[...] = _softmax(q)
        p = lax.dot_general(w_e, W, dn, preferred_element_type=jnp.float32)
        P_ref[...] = _softmax(p)

        a = (q + g_ref[...]) / tau_ref[0, 0]
        m = jnp.max(a, axis=1, keepdims=True)
        iota = lax.broadcasted_iota(jnp.int32, a.shape, 1)
        ksel = jnp.min(jnp.where(a >= m, iota, CAT), axis=1, keepdims=True)
        H = (iota == ksel).astype(jnp.float32)

        Cc = lax.dot_general(cc_ref[...], W, dn,
                             preferred_element_type=jnp.float32)
        pos = _logsig(jnp.sum(H * Cc, axis=1, keepdims=True))
        V = lax.dot_general(H, W, (((1,), (0,)), ((), ())),
                            preferred_element_type=jnp.float32)
        acc = jnp.zeros((R, 1), jnp.float32)
        for j in range(nneg):
            s = jnp.sum(neg_ref[j] * V, axis=1, keepdims=True)
            acc = acc + _logsig(-s)
        L_ref[0, 0, 0] = jnp.sum(pos + acc * (1.0 / nneg))

    return pl.pallas_call(
        body,
        grid=(grid,),
        in_specs=[
            pl.BlockSpec((R, DIM), lambda i: (i, 0)),
            pl.BlockSpec((R, DIM), lambda i: (i, 0)),
            pl.BlockSpec((R, DIM), lambda i: (i, 0)),
            pl.BlockSpec((nneg, R, DIM), lambda i: (0, i, 0)),
            pl.BlockSpec((R, CAT), lambda i: (i, 0)),
            pl.BlockSpec(memory_space=pltpu.SMEM),
            pl.BlockSpec((CAT, DIM), lambda i: (0, 0)),
        ],
        out_specs=[
            pl.BlockSpec((R, CAT), lambda i: (i, 0)),
            pl.BlockSpec((R, CAT), lambda i: (i, 0)),
            pl.BlockSpec((1, 1, 1), lambda i: (i, 0, 0),
                         memory_space=pltpu.SMEM),
        ],
        out_shape=[
            jax.ShapeDtypeStruct((B, CAT), jnp.float32),
            jax.ShapeDtypeStruct((B, CAT), jnp.float32),
            jax.ShapeDtypeStruct((grid, 1, 1), jnp.float32),
        ],
    )(w_e, c_e, c_ctx, negT, g, tau, W_comm)


def kernel(w, c, neg, temp, node_emb, ctx_emb, W_comm):
    B = w.shape[0]
    nneg = neg.shape[1]
    nidx = jnp.concatenate([w, c])
    cidx = jnp.concatenate([c, neg.T.reshape(-1)])
    node_rows, ctx_rows = _sc_gather(node_emb, ctx_emb, nidx, cidx)
    w_e = node_rows[:B]
    c_e = node_rows[B:]
    c_ctx = ctx_rows[:B]
    negT = ctx_rows[B:].reshape(nneg, B, DIM)

    u = jax.random.uniform(jax.random.key(42), (B, CAT),
                           minval=1e-10, maxval=1.0)
    g = -jnp.log(-jnp.log(u))
    tau = jnp.asarray(temp, jnp.float32).reshape(1, 1)

    S, P, Lp = _dense_tc(w_e, c_e, c_ctx, negT, g, tau, W_comm)
    loss = -jnp.sum(Lp) / B
    return (loss, S, P)


# speedup vs baseline: 1.1036x; 1.0881x over previous
import functools

import jax
import jax.numpy as jnp
from jax import lax
from jax.experimental import pallas as pl
from jax.experimental.pallas import tpu as pltpu
from jax.experimental.pallas import tpu_sc as plsc

DIM = 64
CAT = 32
NC, NS = 2, 16
NW = NC * NS
CHUNK = 512
SUB = 128
RING = 3


def _sc_gather(node_emb, ctx_emb, w, c, negf):
    dim = node_emb.shape[1]
    B = w.shape[0]
    NF = negf.shape[0]
    b_per = B // NW
    nf_per = NF // NW
    assert B % NW == 0 and NF % NW == 0
    assert b_per % CHUNK == 0 and nf_per % CHUNK == 0

    mesh = plsc.VectorSubcoreMesh(
        core_axis_name="c", subcore_axis_name="s",
        num_cores=NC, num_subcores=NS)

    @functools.partial(
        pl.kernel,
        mesh=mesh,
        out_type=(
            jax.ShapeDtypeStruct((B, dim), jnp.float32),
            jax.ShapeDtypeStruct((B, dim), jnp.float32),
            jax.ShapeDtypeStruct((B, dim), jnp.float32),
            jax.ShapeDtypeStruct((NF, dim), jnp.float32),
        ),
        scratch_types=[
            pltpu.VMEM((b_per,), jnp.int32),
            pltpu.VMEM((b_per,), jnp.int32),
            pltpu.VMEM((nf_per,), jnp.int32),
            pltpu.VMEM((RING, CHUNK, dim), jnp.float32),
            pltpu.SemaphoreType.DMA((RING,)),
            pltpu.SemaphoreType.DMA((RING,)),
        ],
        compiler_params=pltpu.CompilerParams(use_tc_tiling_on_sc=False),
    )
    def gather_kernel(node_hbm, ctx_hbm, w_hbm, c_hbm, negf_hbm,
                      we_hbm, ce_hbm, cc_hbm, neg_hbm,
                      widx_v, cidx_v, nidx_v, rows_v, gsem, wsem):
        wid = lax.axis_index("s") * NC + lax.axis_index("c")
        wb = wid * b_per
        nb = wid * nf_per
        pltpu.sync_copy(w_hbm.at[pl.ds(wb, b_per)], widx_v)
        pltpu.sync_copy(c_hbm.at[pl.ds(wb, b_per)], cidx_v)
        pltpu.sync_copy(negf_hbm.at[pl.ds(nb, nf_per)], nidx_v)

        jobs = []
        for k in range(b_per // CHUNK):
            jobs.append((node_hbm, widx_v, k * CHUNK, we_hbm, wb + k * CHUNK))
            jobs.append((node_hbm, cidx_v, k * CHUNK, ce_hbm, wb + k * CHUNK))
            jobs.append((ctx_hbm, cidx_v, k * CHUNK, cc_hbm, wb + k * CHUNK))
        for k in range(nf_per // CHUNK):
            jobs.append((ctx_hbm, nidx_v, k * CHUNK, neg_hbm, nb + k * CHUNK))
        njobs = len(jobs)

        pend_g = [None] * RING
        pend_w = [None] * RING

        def fire(j):
            p = j % RING
            if pend_w[p] is not None:
                pend_w[p].wait()
                pend_w[p] = None
            table, idx_ref, ioff, _, _ = jobs[j]
            pend_g[p] = [
                pltpu.async_copy(
                    table.at[idx_ref.at[pl.ds(ioff + k * SUB, SUB)]],
                    rows_v.at[p].at[pl.ds(k * SUB, SUB)],
                    gsem.at[p])
                for k in range(CHUNK // SUB)
            ]

        for j in range(min(RING - 1, njobs)):
            fire(j)
        for j in range(njobs):
            if j + RING - 1 < njobs:
                fire(j + RING - 1)
            p = j % RING
            for d in pend_g[p]:
                d.wait()
            _, _, _, out, ooff = jobs[j]
            pend_w[p] = pltpu.async_copy(
                rows_v.at[p], out.at[pl.ds(ooff, CHUNK)], wsem.at[p])
        for p in range(RING):
            if pend_w[p] is not None:
                pend_w[p].wait()

    return gather_kernel(node_emb, ctx_emb, w, c, negf)


def _softmax(x):
    m = jnp.max(x, axis=1, keepdims=True)
    e = jnp.exp(x - m)
    return e / jnp.sum(e, axis=1, keepdims=True)


def _logsig(x):
    return jnp.minimum(x, 0.0) - jnp.log(1.0 + jnp.exp(-jnp.abs(x)))


def _dense_tc(w_e, c_e, c_ctx, negT, g, tau, W_comm):
    B = w_e.shape[0]
    nneg = negT.shape[0]
    R = 1024
    grid = B // R
    dn = (((1,), (1,)), ((), ()))

    def body(w_ref, c_ref, cc_ref, neg_ref, g_ref, tau_ref, W_ref,
             S_ref, P_ref, L_ref):
        W = W_ref[...]
        w_e = w_ref[...]
        pw = w_e * c_ref[...]
        q = lax.dot_general(pw, W, dn, preferred_element_type=jnp.float32)
        S_ref[...] = _softmax(q)
        p = lax.dot_general(w_e, W, dn, preferred_element_type=jnp.float32)
        P_ref[...] = _softmax(p)

        a = (q + g_ref[...]) / tau_ref[0, 0]
        m = jnp.max(a, axis=1, keepdims=True)
        iota = lax.broadcasted_iota(jnp.int32, a.shape, 1)
        ksel = jnp.min(jnp.where(a >= m, iota, CAT), axis=1, keepdims=True)
        H = (iota == ksel).astype(jnp.float32)

        Cc = lax.dot_general(cc_ref[...], W, dn,
                             preferred_element_type=jnp.float32)
        pos = _logsig(jnp.sum(H * Cc, axis=1, keepdims=True))
        V = lax.dot_general(H, W, (((1,), (0,)), ((), ())),
                            preferred_element_type=jnp.float32)
        acc = jnp.zeros((R, 1), jnp.float32)
        for j in range(nneg):
            s = jnp.sum(neg_ref[j] * V, axis=1, keepdims=True)
            acc = acc + _logsig(-s)
        L_ref[0, 0, 0] = jnp.sum(pos + acc * (1.0 / nneg))

    return pl.pallas_call(
        body,
        grid=(grid,),
        in_specs=[
            pl.BlockSpec((R, DIM), lambda i: (i, 0)),
            pl.BlockSpec((R, DIM), lambda i: (i, 0)),
            pl.BlockSpec((R, DIM), lambda i: (i, 0)),
            pl.BlockSpec((nneg, R, DIM), lambda i: (0, i, 0)),
            pl.BlockSpec((R, CAT), lambda i: (i, 0)),
            pl.BlockSpec(memory_space=pltpu.SMEM),
            pl.BlockSpec((CAT, DIM), lambda i: (0, 0)),
        ],
        out_specs=[
            pl.BlockSpec((R, CAT), lambda i: (i, 0)),
            pl.BlockSpec((R, CAT), lambda i: (i, 0)),
            pl.BlockSpec((1, 1, 1), lambda i: (i, 0, 0),
                         memory_space=pltpu.SMEM),
        ],
        out_shape=[
            jax.ShapeDtypeStruct((B, CAT), jnp.float32),
            jax.ShapeDtypeStruct((B, CAT), jnp.float32),
            jax.ShapeDtypeStruct((grid, 1, 1), jnp.float32),
        ],
    )(w_e, c_e, c_ctx, negT, g, tau, W_comm)


def kernel(w, c, neg, temp, node_emb, ctx_emb, W_comm):
    B = w.shape[0]
    nneg = neg.shape[1]
    negf = neg.T.reshape(-1)
    w_e, c_e, c_ctx, neg_rows = _sc_gather(node_emb, ctx_emb, w, c, negf)
    negT = neg_rows.reshape(nneg, B, DIM)

    u = jax.random.uniform(jax.random.key(42), (B, CAT),
                           minval=1e-10, maxval=1.0)
    g = -jnp.log(-jnp.log(u))
    tau = jnp.asarray(temp, jnp.float32).reshape(1, 1)

    S, P, Lp = _dense_tc(w_e, c_e, c_ctx, negT, g, tau, W_comm)
    loss = -jnp.sum(Lp) / B
    return (loss, S, P)
